# speedup vs baseline: 12.6600x; 12.6600x over previous
import functools
import numpy as np
import jax
import jax.numpy as jnp
from jax import lax
from jax.experimental import pallas as pl
from jax.experimental.pallas import tpu as pltpu
from jax.experimental.pallas import tpu_sc as plsc

_N0, _E0, _N1, _E1 = 10000, 320000, 1250, 40000
_C, _G, _NET = 128, 32, 7
_NBLK = 2
_PN0, _PN1 = 10240, 1280
_CH = 128
_NC, _NS = 2, 16
_NW = _NC * _NS
_EPS = 1e-5

def _pad_edges(e):
    return ((e + _NW * _CH - 1) // (_NW * _CH)) * _NW * _CH

_PE0, _PE1 = _pad_edges(_E0), _pad_edges(_E1)

_MAVG_NP = np.zeros((_C, _C), np.float32)
for _g in range(_G):
    _MAVG_NP[_g * 4:(_g + 1) * 4, _g * 4:(_g + 1) * 4] = 0.25


def _gn_gelu_mm(t, gamma, beta, mavg, w):
    m = jnp.dot(t, mavg, preferred_element_type=jnp.float32)
    msq = jnp.dot(t * t, mavg, preferred_element_type=jnp.float32)
    var = msq - m * m
    xn = (t - m) * lax.rsqrt(var + _EPS)
    h = jax.nn.gelu(xn * gamma + beta)
    return jnp.dot(h, w, preferred_element_type=jnp.float32)


def _dense_first(x, gamma, beta, wcat, *, bn):
    PN = x.shape[0]
    wout = wcat.shape[1]

    def body(x_ref, g_ref, b_ref, m_ref, w_ref, o_ref):
        o_ref[...] = _gn_gelu_mm(x_ref[...], g_ref[...], b_ref[...],
                                 m_ref[...], w_ref[...])

    return pl.pallas_call(
        body,
        grid=(PN // bn,),
        in_specs=[
            pl.BlockSpec((bn, _C), lambda i: (i, 0)),
            pl.BlockSpec((1, _C), lambda i: (0, 0)),
            pl.BlockSpec((1, _C), lambda i: (0, 0)),
            pl.BlockSpec((_C, _C), lambda i: (0, 0)),
            pl.BlockSpec((_C, wout), lambda i: (0, 0)),
        ],
        out_specs=pl.BlockSpec((bn, wout), lambda i: (i, 0)),
        out_shape=jax.ShapeDtypeStruct((PN, wout), jnp.float32),
    )(x, gamma, beta, jnp.asarray(_MAVG_NP), wcat)


def _dense_mid(p0, p1, bias, gamma, beta, wcat, *, bn):
    PN = p0.shape[0]
    wout = wcat.shape[1]

    def body(p0_ref, p1_ref, bias_ref, g_ref, b_ref, m_ref, w_ref, o_ref):
        t = p0_ref[...] + p1_ref[...] + bias_ref[...]
        o_ref[...] = _gn_gelu_mm(t, g_ref[...], b_ref[...],
                                 m_ref[...], w_ref[...])

    return pl.pallas_call(
        body,
        grid=(PN // bn,),
        in_specs=[
            pl.BlockSpec((bn, _C), lambda i: (i, 0)),
            pl.BlockSpec((bn, _C), lambda i: (i, 0)),
            pl.BlockSpec((1, _C), lambda i: (0, 0)),
            pl.BlockSpec((1, _C), lambda i: (0, 0)),
            pl.BlockSpec((1, _C), lambda i: (0, 0)),
            pl.BlockSpec((_C, _C), lambda i: (0, 0)),
            pl.BlockSpec((_C, wout), lambda i: (0, 0)),
        ],
        out_specs=pl.BlockSpec((bn, wout), lambda i: (i, 0)),
        out_shape=jax.ShapeDtypeStruct((PN, wout), jnp.float32),
    )(p0, p1, bias, gamma, beta, jnp.asarray(_MAVG_NP), wcat)


def _dense_ds(s0, s1, c0, c1, gamma, beta, w, bias_out, *, bn):
    PN = s0.shape[0]

    def body(s0_ref, s1_ref, c0_ref, c1_ref, g_ref, b_ref, m_ref, w_ref,
             bo_ref, o_ref):
        cnt = jnp.maximum(c0_ref[...] + c1_ref[...], 1.0)
        t = (s0_ref[...] + s1_ref[...]) / cnt
        o_ref[...] = _gn_gelu_mm(t, g_ref[...], b_ref[...], m_ref[...],
                                 w_ref[...]) + bo_ref[...]

    return pl.pallas_call(
        body,
        grid=(PN // bn,),
        in_specs=[
            pl.BlockSpec((bn, _C), lambda i: (i, 0)),
            pl.BlockSpec((bn, _C), lambda i: (i, 0)),
            pl.BlockSpec((bn, _C), lambda i: (i, 0)),
            pl.BlockSpec((bn, _C), lambda i: (i, 0)),
            pl.BlockSpec((1, _C), lambda i: (0, 0)),
            pl.BlockSpec((1, _C), lambda i: (0, 0)),
            pl.BlockSpec((_C, _C), lambda i: (0, 0)),
            pl.BlockSpec((_C, _C), lambda i: (0, 0)),
            pl.BlockSpec((1, _C), lambda i: (0, 0)),
        ],
        out_specs=pl.BlockSpec((bn, _C), lambda i: (i, 0)),
        out_shape=jax.ShapeDtypeStruct((PN, _C), jnp.float32),
    )(s0, s1, c0, c1, gamma, beta, jnp.asarray(_MAVG_NP), w, bias_out)


def _resadd(x, p0, p1, bias, *, bn):
    PN = x.shape[0]

    def body(x_ref, p0_ref, p1_ref, b_ref, o_ref):
        o_ref[...] = x_ref[...] + p0_ref[...] + p1_ref[...] + b_ref[...]

    return pl.pallas_call(
        body,
        grid=(PN // bn,),
        in_specs=[
            pl.BlockSpec((bn, _C), lambda i: (i, 0)),
            pl.BlockSpec((bn, _C), lambda i: (i, 0)),
            pl.BlockSpec((bn, _C), lambda i: (i, 0)),
            pl.BlockSpec((1, _C), lambda i: (0, 0)),
        ],
        out_specs=pl.BlockSpec((bn, _C), lambda i: (i, 0)),
        out_shape=jax.ShapeDtypeStruct((PN, _C), jnp.float32),
    )(x, p0, p1, bias)


def _make_agg(PN, PE):
    EPT = PE // _NW
    NCHUNKS = EPT // _CH
    ZR = PN // _NS
    mesh = plsc.VectorSubcoreMesh(core_axis_name="c", subcore_axis_name="s")

    @functools.partial(
        pl.kernel,
        out_type=jax.ShapeDtypeStruct((_NC, PN, _C), jnp.float32),
        mesh=mesh,
        scratch_types=[
            pltpu.VMEM_SHARED((PN, _C), jnp.float32),
            pltpu.VMEM((1, _CH), jnp.int32),
            pltpu.VMEM((1, _CH), jnp.int32),
            pltpu.VMEM((_CH, _C), jnp.float32),
            pltpu.SemaphoreType.DMA,
        ],
    )
    def agg(y_hbm, rows_hbm, dst_hbm, zeros_hbm, out_hbm,
            accum, ridx, didx, rbuf, gsem):
        cid = lax.axis_index("c")
        sid = lax.axis_index("s")
        wid = sid * _NC + cid
        pltpu.sync_copy(zeros_hbm.at[pl.ds(sid * ZR, ZR)],
                        accum.at[pl.ds(sid * ZR, ZR)])
        plsc.subcore_barrier()
        ebase = wid * EPT

        @pl.loop(0, NCHUNKS)
        def _chunk(j):
            off = ebase + j * _CH
            pltpu.sync_copy(rows_hbm.at[pl.ds(off, _CH)], ridx.at[0])
            pltpu.sync_copy(dst_hbm.at[pl.ds(off, _CH)], didx.at[0])
            pltpu.async_copy(y_hbm.at[ridx.at[0]], rbuf, gsem).wait()
            pltpu.sync_copy(rbuf, accum.at[didx.at[0]], add=True)

        plsc.subcore_barrier()
        pltpu.sync_copy(accum.at[pl.ds(sid * ZR, ZR)],
                        out_hbm.at[cid, pl.ds(sid * ZR, ZR)])

    return agg


def _make_pool(PNSRC, PNDST):
    RPT = PNSRC // _NW
    PCH = 64
    NCHUNKS = RPT // PCH
    ZR = PNDST // _NS
    mesh = plsc.VectorSubcoreMesh(core_axis_name="c", subcore_axis_name="s")

    @functools.partial(
        pl.kernel,
        out_type=(jax.ShapeDtypeStruct((_NC, PNDST, _C), jnp.float32),
                  jax.ShapeDtypeStruct((_NC, PNDST, _C), jnp.float32)),
        mesh=mesh,
        scratch_types=[
            pltpu.VMEM_SHARED((PNDST, _C), jnp.float32),
            pltpu.VMEM_SHARED((PNDST, _C), jnp.float32),
            pltpu.VMEM((1, PCH), jnp.int32),
            pltpu.VMEM((PCH, _C), jnp.float32),
            pltpu.VMEM((PCH, _C), jnp.float32),
        ],
    )
    def pool(x_hbm, parent_hbm, zeros_hbm, ones_hbm, sum_hbm, cnt_hbm,
             saccum, caccum, pidx, rbuf, obuf):
        cid = lax.axis_index("c")
        sid = lax.axis_index("s")
        wid = sid * _NC + cid
        pltpu.sync_copy(zeros_hbm.at[pl.ds(sid * ZR, ZR)],
                        saccum.at[pl.ds(sid * ZR, ZR)])
        pltpu.sync_copy(zeros_hbm.at[pl.ds(sid * ZR, ZR)],
                        caccum.at[pl.ds(sid * ZR, ZR)])
        pltpu.sync_copy(ones_hbm, obuf)
        plsc.subcore_barrier()
        rbase = wid * RPT

        @pl.loop(0, NCHUNKS)
        def _chunk(j):
            off = rbase + j * PCH
            pltpu.sync_copy(parent_hbm.at[pl.ds(off, PCH)], pidx.at[0])
            pltpu.sync_copy(x_hbm.at[pl.ds(off, PCH)], rbuf)
            pltpu.sync_copy(rbuf, saccum.at[pidx.at[0]], add=True)
            pltpu.sync_copy(obuf, caccum.at[pidx.at[0]], add=True)

        plsc.subcore_barrier()
        pltpu.sync_copy(saccum.at[pl.ds(sid * ZR, ZR)],
                        sum_hbm.at[cid, pl.ds(sid * ZR, ZR)])
        pltpu.sync_copy(caccum.at[pl.ds(sid * ZR, ZR)],
                        cnt_hbm.at[cid, pl.ds(sid * ZR, ZR)])

    return pool


_agg0 = _make_agg(_PN0, _PE0)
_agg1 = _make_agg(_PN1, _PE1)
_pool = _make_pool(_PN0, _PN1)


def _stage(x, rows, dst, zeros, agg, gn_gamma, gn_beta, wcats, bc, *, bn):
    for blk in range(_NBLK):
        y = _dense_first(x, gn_gamma[blk, 0:1, :], gn_beta[blk, 0:1, :],
                         wcats[blk][0], bn=bn)
        p = agg(y.reshape(-1, _C), rows, dst, zeros)
        y = _dense_mid(p[0], p[1], bc[blk, 0:1, :],
                       gn_gamma[blk, 1:2, :], gn_beta[blk, 1:2, :],
                       wcats[blk][1], bn=bn)
        p = agg(y.reshape(-1, _C), rows, dst, zeros)
        x = _resadd(x, p[0], p[1], bc[blk, 1:2, :], bn=bn)
    return x


def kernel(data, edge_index0, edge_type0, parent_idx, edge_index1, edge_type1,
           depth, gn_gamma, gn_beta, Wc, bc, ds_gamma, ds_beta, W_ds, b_ds):
    del depth
    f32 = jnp.float32

    rows0 = edge_index0[0] * _NET + edge_type0
    rows0 = jnp.concatenate([rows0, jnp.zeros((_PE0 - _E0,), jnp.int32)])
    dst0 = jnp.concatenate([edge_index0[1],
                            jnp.full((_PE0 - _E0,), _N0, jnp.int32)])
    rows1 = edge_index1[0] * _NET + edge_type1
    rows1 = jnp.concatenate([rows1, jnp.zeros((_PE1 - _E1,), jnp.int32)])
    dst1 = jnp.concatenate([edge_index1[1],
                            jnp.full((_PE1 - _E1,), _N1, jnp.int32)])
    parent = jnp.concatenate([parent_idx,
                              jnp.full((_PN0 - _N0,), _PN1 - 1, jnp.int32)])
    xp = jnp.concatenate([data, jnp.zeros((_PN0 - _N0, _C), f32)])
    zeros0 = jnp.zeros((_PN0, _C), f32)
    zeros1 = jnp.zeros((_PN1, _C), f32)
    ones64 = jnp.ones((64, _C), f32)

    wcats = [[[jnp.transpose(Wc[s, blk, i], (1, 0, 2)).reshape(_C, _NET * _C)
               for i in range(2)] for blk in range(_NBLK)]
             for s in range(2)]

    out_d = _stage(xp, rows0, dst0, zeros0, _agg0,
                   gn_gamma[0], gn_beta[0], wcats[0], bc[0], bn=512)

    sums, cnts = _pool(out_d, parent, zeros1, ones64)
    x1 = _dense_ds(sums[0], sums[1], cnts[0], cnts[1],
                   ds_gamma.reshape(1, _C), ds_beta.reshape(1, _C),
                   W_ds, b_ds.reshape(1, _C), bn=256)

    out_dm1 = _stage(x1, rows1, dst1, zeros1, _agg1,
                     gn_gamma[1], gn_beta[1], wcats[1], bc[1], bn=256)

    return (out_d[:_N0], out_dm1[:_N1])
